# Initial kernel scaffold; baseline (speedup 1.0000x reference)
import jax
import jax.numpy as jnp
from jax.experimental import pallas as pl


def kernel(inputs, codewords):
    raise NotImplementedError("write your pallas kernel here")


# speedup vs baseline: 2.5440x; 2.5440x over previous
import functools

import jax
import jax.numpy as jnp
from jax import lax
from jax.experimental import pallas as pl
from jax.experimental.pallas import tpu as pltpu
from jax.experimental.pallas import tpu_sc as plsc

_K = 512
_D = 32
_CHUNK = 1024


def _tree_sum_minor(v):
    groups = []
    for g in range(4):
        b = v[:, 8 * g:8 * g + 8]
        b = b[:, :4] + b[:, 4:]
        b = b[:, :2] + b[:, 2:]
        b = b[:, :1] + b[:, 1:]
        groups.append(b)
    return (groups[0] + groups[1]) + (groups[2] + groups[3])


def _tc_body(x_ref, c_ref, ct_ref, idx_ref, loss_ref):
    x = x_ref[...]
    c = c_ref[...]
    ct = ct_ref[...]
    cn = jnp.sum(ct * ct, axis=0, keepdims=True)
    xc = lax.dot_general(x, ct, (((1,), (0,)), ((), ())),
                         preferred_element_type=jnp.float32,
                         precision=lax.Precision.HIGHEST)
    s = cn - 2.0 * xc
    iota = lax.broadcasted_iota(jnp.int32, s.shape, 1)
    idx1 = jnp.argmin(s, axis=1).astype(jnp.int32)
    m1 = idx1[:, None] == iota
    s2 = jnp.where(m1, jnp.float32(1e30), s)
    idx2 = jnp.argmin(s2, axis=1).astype(jnp.int32)
    m2 = idx2[:, None] == iota
    cdims = (((1,), (0,)), ((), ()))
    c1 = lax.dot_general(m1.astype(jnp.float32), c, cdims,
                         preferred_element_type=jnp.float32,
                         precision=lax.Precision.HIGHEST)
    c2 = lax.dot_general(m2.astype(jnp.float32), c, cdims,
                         preferred_element_type=jnp.float32,
                         precision=lax.Precision.HIGHEST)
    r1 = x - c1
    d1 = _tree_sum_minor(r1 * r1)
    r2 = x - c2
    d2 = _tree_sum_minor(r2 * r2)
    take1 = (d1 < d2) | ((d1 == d2) & (idx1[:, None] < idx2[:, None]))
    win = jnp.where(take1[:, 0], idx1, idx2)
    dmin = jnp.where(take1, d1, d2)
    idx_ref[...] = win[:, None]
    part = jnp.sum(dmin, axis=0, keepdims=True) * jnp.float32(2.0 ** -20)

    @pl.when(pl.program_id(0) == 0)
    def _():
        loss_ref[...] = jnp.zeros_like(loss_ref[...])

    loss_ref[...] += part


def _sc_gather(table, idx):
    info = plsc.get_sparse_core_info()
    nw = info.num_cores * info.num_subcores
    n = idx.shape[0]
    bpw = n // nw
    mesh = plsc.VectorSubcoreMesh(core_axis_name="c", subcore_axis_name="s")

    @functools.partial(
        pl.kernel, mesh=mesh,
        out_type=jax.ShapeDtypeStruct((n, _D), jnp.float32),
        compiler_params=pltpu.CompilerParams(use_tc_tiling_on_sc=False),
        scratch_types=[
            pltpu.VMEM((bpw,), jnp.int32),
            pltpu.VMEM((bpw, _D), jnp.float32),
            pltpu.SemaphoreType.DMA,
        ],
    )
    def gather_kernel(table_hbm, idx_hbm, out_hbm, idx_v, rows_v, sem):
        wid = lax.axis_index("s") * info.num_cores + lax.axis_index("c")
        base = wid * bpw
        pltpu.sync_copy(idx_hbm.at[pl.ds(base, bpw)], idx_v)
        pltpu.async_copy(table_hbm.at[idx_v], rows_v, sem).wait()
        pltpu.sync_copy(rows_v, out_hbm.at[pl.ds(base, bpw)])

    return gather_kernel(table, idx)


def kernel(inputs, codewords):
    shape = inputs.shape
    n = shape[0] * shape[1]
    x = inputs.reshape(n, _D)
    ct = codewords.T
    idx2d, loss2d = pl.pallas_call(
        _tc_body,
        grid=(n // _CHUNK,),
        in_specs=[
            pl.BlockSpec((_CHUNK, _D), lambda i: (i, 0)),
            pl.BlockSpec((_K, _D), lambda i: (0, 0)),
            pl.BlockSpec((_D, _K), lambda i: (0, 0)),
        ],
        out_specs=[
            pl.BlockSpec((_CHUNK, 1), lambda i: (i, 0)),
            pl.BlockSpec((1, 1), lambda i: (0, 0)),
        ],
        out_shape=[
            jax.ShapeDtypeStruct((n, 1), jnp.int32),
            jax.ShapeDtypeStruct((1, 1), jnp.float32),
        ],
    )(x, codewords, ct)
    idx_flat = idx2d.reshape(n)
    quantized = _sc_gather(codewords, idx_flat).reshape(shape)
    indices = idx2d.reshape(shape[:-1])
    loss = loss2d[0, 0]
    return quantized, indices, loss


# speedup vs baseline: 3.2640x; 1.2830x over previous
import functools

import jax
import jax.numpy as jnp
from jax import lax
from jax.experimental import pallas as pl
from jax.experimental.pallas import tpu as pltpu
from jax.experimental.pallas import tpu_sc as plsc

_K = 512
_D = 32
_CHUNK = 1024


def _tree_sum_minor(v):
    groups = []
    for g in range(4):
        b = v[:, 8 * g:8 * g + 8]
        b = b[:, :4] + b[:, 4:]
        b = b[:, :2] + b[:, 2:]
        b = b[:, :1] + b[:, 1:]
        groups.append(b)
    return (groups[0] + groups[1]) + (groups[2] + groups[3])


def _tc_body(x_ref, c_ref, ct_ref, idx_ref, loss_ref, q_ref):
    x = x_ref[...]
    c = c_ref[...]
    ct = ct_ref[...]
    cn = jnp.sum(ct * ct, axis=0, keepdims=True)
    xc = lax.dot_general(x, ct, (((1,), (0,)), ((), ())),
                         preferred_element_type=jnp.float32,
                         precision=lax.Precision.HIGHEST)
    s = cn - 2.0 * xc
    iota = lax.broadcasted_iota(jnp.int32, s.shape, 1)
    idx1 = jnp.argmin(s, axis=1).astype(jnp.int32)
    m1 = idx1[:, None] == iota
    s2 = jnp.where(m1, jnp.float32(1e30), s)
    idx2 = jnp.argmin(s2, axis=1).astype(jnp.int32)
    m2 = idx2[:, None] == iota
    cdims = (((1,), (0,)), ((), ()))
    c1 = lax.dot_general(m1.astype(jnp.float32), c, cdims,
                         preferred_element_type=jnp.float32,
                         precision=lax.Precision.HIGHEST)
    c2 = lax.dot_general(m2.astype(jnp.float32), c, cdims,
                         preferred_element_type=jnp.float32,
                         precision=lax.Precision.HIGHEST)
    r1 = x - c1
    d1 = _tree_sum_minor(r1 * r1)
    r2 = x - c2
    d2 = _tree_sum_minor(r2 * r2)
    take1 = (d1 < d2) | ((d1 == d2) & (idx1[:, None] < idx2[:, None]))
    win = jnp.where(take1[:, 0], idx1, idx2)
    dmin = jnp.where(take1, d1, d2)
    idx_ref[...] = win[:, None]
    q_ref[...] = jnp.where(take1, c1, c2)
    part = jnp.sum(dmin, axis=0, keepdims=True) * jnp.float32(2.0 ** -20)

    @pl.when(pl.program_id(0) == 0)
    def _():
        loss_ref[...] = jnp.zeros_like(loss_ref[...])

    loss_ref[...] += part


def _sc_gather(table, idx):
    info = plsc.get_sparse_core_info()
    nw = info.num_cores * info.num_subcores
    n = idx.shape[0]
    bpw = n // nw
    mesh = plsc.VectorSubcoreMesh(core_axis_name="c", subcore_axis_name="s")

    @functools.partial(
        pl.kernel, mesh=mesh,
        out_type=jax.ShapeDtypeStruct((n, _D), jnp.float32),
        compiler_params=pltpu.CompilerParams(use_tc_tiling_on_sc=False),
        scratch_types=[
            pltpu.VMEM((bpw,), jnp.int32),
            pltpu.VMEM((bpw, _D), jnp.float32),
            pltpu.SemaphoreType.DMA,
        ],
    )
    def gather_kernel(table_hbm, idx_hbm, out_hbm, idx_v, rows_v, sem):
        wid = lax.axis_index("s") * info.num_cores + lax.axis_index("c")
        base = wid * bpw
        pltpu.sync_copy(idx_hbm.at[pl.ds(base, bpw)], idx_v)
        pltpu.async_copy(table_hbm.at[idx_v], rows_v, sem).wait()
        pltpu.sync_copy(rows_v, out_hbm.at[pl.ds(base, bpw)])

    return gather_kernel(table, idx)


def kernel(inputs, codewords):
    shape = inputs.shape
    n = shape[0] * shape[1]
    x = inputs.reshape(n, _D)
    ct = codewords.T
    idx2d, loss2d, q2d = pl.pallas_call(
        _tc_body,
        grid=(n // _CHUNK,),
        in_specs=[
            pl.BlockSpec((_CHUNK, _D), lambda i: (i, 0)),
            pl.BlockSpec((_K, _D), lambda i: (0, 0)),
            pl.BlockSpec((_D, _K), lambda i: (0, 0)),
        ],
        out_specs=[
            pl.BlockSpec((_CHUNK, 1), lambda i: (i, 0)),
            pl.BlockSpec((1, 1), lambda i: (0, 0)),
            pl.BlockSpec((_CHUNK, _D), lambda i: (i, 0)),
        ],
        out_shape=[
            jax.ShapeDtypeStruct((n, 1), jnp.int32),
            jax.ShapeDtypeStruct((1, 1), jnp.float32),
            jax.ShapeDtypeStruct((n, _D), jnp.float32),
        ],
    )(x, codewords, ct)
    quantized = q2d.reshape(shape)
    indices = idx2d.reshape(shape[:-1])
    loss = loss2d[0, 0]
    return quantized, indices, loss


# speedup vs baseline: 3.7068x; 1.1357x over previous
import functools

import jax
import jax.numpy as jnp
from jax import lax
from jax.experimental import pallas as pl
from jax.experimental.pallas import tpu as pltpu
from jax.experimental.pallas import tpu_sc as plsc

_K = 512
_D = 32
_CHUNK = 2048


def _tree_sum_minor(v):
    groups = []
    for g in range(4):
        b = v[:, 8 * g:8 * g + 8]
        b = b[:, :4] + b[:, 4:]
        b = b[:, :2] + b[:, 2:]
        b = b[:, :1] + b[:, 1:]
        groups.append(b)
    return (groups[0] + groups[1]) + (groups[2] + groups[3])


def _tc_body(x_ref, c_ref, ct_ref, idx_ref, loss_ref, q_ref):
    cc = x_ref.shape[0]
    x = x_ref[...]
    c = c_ref[...]
    ct = ct_ref[...]
    cn_half = 0.5 * jnp.sum(ct * ct, axis=0, keepdims=True)
    cdims = (((1,), (0,)), ((), ()))

    def mm(a, b):
        return lax.dot_general(a, b, cdims,
                               preferred_element_type=jnp.float32)

    xh = x.astype(jnp.bfloat16)
    xl = (x - xh.astype(jnp.float32)).astype(jnp.bfloat16)
    cth = ct.astype(jnp.bfloat16)
    ctl = (ct - cth.astype(jnp.float32)).astype(jnp.bfloat16)
    xc = mm(xh, cth) + (mm(xh, ctl) + mm(xl, cth))
    s = cn_half - xc
    iota = lax.broadcasted_iota(jnp.int32, s.shape, 1)
    big = jnp.float32(3e38)
    kbig = jnp.int32(_K)
    m1v = jnp.min(s, axis=1, keepdims=True)
    idx1 = jnp.min(jnp.where(s == m1v, iota, kbig), axis=1, keepdims=True)
    first = iota == idx1
    m2v = jnp.min(jnp.where(first, big, s), axis=1, keepdims=True)
    idx2 = jnp.min(jnp.where((s == m2v) & jnp.logical_not(first), iota, kbig),
                   axis=1, keepdims=True)
    idx12 = jnp.concatenate([idx1, idx2], axis=0)
    iota2 = lax.broadcasted_iota(jnp.int32, (2 * cc, _K), 1)
    oh12 = (iota2 == idx12).astype(jnp.bfloat16)
    ch = c.astype(jnp.bfloat16)
    cmf = c - ch.astype(jnp.float32)
    cm = cmf.astype(jnp.bfloat16)
    cl = (cmf - cm.astype(jnp.float32)).astype(jnp.bfloat16)
    c12 = (mm(oh12, ch) + mm(oh12, cm)) + mm(oh12, cl)
    x12 = jnp.concatenate([x, x], axis=0)
    r = x12 - c12
    d12 = _tree_sum_minor(r * r)
    d1 = d12[:cc]
    d2 = d12[cc:]
    take1 = (d1 < d2) | ((d1 == d2) & (idx1 < idx2))
    win = jnp.where(take1, idx1, idx2)
    dmin = jnp.where(take1, d1, d2)
    idx_ref[...] = win
    q_ref[...] = jnp.where(take1, c12[:cc], c12[cc:])
    part = jnp.sum(dmin, axis=0, keepdims=True) * jnp.float32(2.0 ** -20)

    @pl.when(pl.program_id(0) == 0)
    def _():
        loss_ref[...] = jnp.zeros_like(loss_ref[...])

    loss_ref[...] += part


def _sc_gather(table, idx):
    info = plsc.get_sparse_core_info()
    nw = info.num_cores * info.num_subcores
    n = idx.shape[0]
    bpw = n // nw
    mesh = plsc.VectorSubcoreMesh(core_axis_name="c", subcore_axis_name="s")

    @functools.partial(
        pl.kernel, mesh=mesh,
        out_type=jax.ShapeDtypeStruct((n, _D), jnp.float32),
        compiler_params=pltpu.CompilerParams(use_tc_tiling_on_sc=False),
        scratch_types=[
            pltpu.VMEM((bpw,), jnp.int32),
            pltpu.VMEM((bpw, _D), jnp.float32),
            pltpu.SemaphoreType.DMA,
        ],
    )
    def gather_kernel(table_hbm, idx_hbm, out_hbm, idx_v, rows_v, sem):
        wid = lax.axis_index("s") * info.num_cores + lax.axis_index("c")
        base = wid * bpw
        pltpu.sync_copy(idx_hbm.at[pl.ds(base, bpw)], idx_v)
        pltpu.async_copy(table_hbm.at[idx_v], rows_v, sem).wait()
        pltpu.sync_copy(rows_v, out_hbm.at[pl.ds(base, bpw)])

    return gather_kernel(table, idx)


def kernel(inputs, codewords):
    shape = inputs.shape
    n = shape[0] * shape[1]
    x = inputs.reshape(n, _D)
    ct = codewords.T
    idx2d, loss2d, q2d = pl.pallas_call(
        _tc_body,
        grid=(n // _CHUNK,),
        in_specs=[
            pl.BlockSpec((_CHUNK, _D), lambda i: (i, 0)),
            pl.BlockSpec((_K, _D), lambda i: (0, 0)),
            pl.BlockSpec((_D, _K), lambda i: (0, 0)),
        ],
        out_specs=[
            pl.BlockSpec((_CHUNK, 1), lambda i: (i, 0)),
            pl.BlockSpec((1, 1), lambda i: (0, 0)),
            pl.BlockSpec((_CHUNK, _D), lambda i: (i, 0)),
        ],
        out_shape=[
            jax.ShapeDtypeStruct((n, 1), jnp.int32),
            jax.ShapeDtypeStruct((1, 1), jnp.float32),
            jax.ShapeDtypeStruct((n, _D), jnp.float32),
        ],
    )(x, codewords, ct)
    quantized = q2d.reshape(shape)
    indices = idx2d.reshape(shape[:-1])
    loss = loss2d[0, 0]
    return quantized, indices, loss


# speedup vs baseline: 4.8610x; 1.3114x over previous
import functools

import jax
import jax.numpy as jnp
from jax import lax
from jax.experimental import pallas as pl
from jax.experimental.pallas import tpu as pltpu
from jax.experimental.pallas import tpu_sc as plsc

_K = 512
_D = 32
_CHUNK = 2048


def _tree_sum_minor(v):
    groups = []
    for g in range(4):
        b = v[:, 8 * g:8 * g + 8]
        b = b[:, :4] + b[:, 4:]
        b = b[:, :2] + b[:, 2:]
        b = b[:, :1] + b[:, 1:]
        groups.append(b)
    return (groups[0] + groups[1]) + (groups[2] + groups[3])


def _tc_body(x_ref, c_ref, ct_ref, idx_ref, loss_ref, q_ref):
    cc = x_ref.shape[0]
    x = x_ref[...]
    c = c_ref[...]
    ct = ct_ref[...]
    cn_half = 0.5 * jnp.sum(ct * ct, axis=0, keepdims=True)
    cdims = (((1,), (0,)), ((), ()))

    def mm(a, b):
        return lax.dot_general(a, b, cdims,
                               preferred_element_type=jnp.float32)

    xh = x.astype(jnp.bfloat16)
    xl = (x - xh.astype(jnp.float32)).astype(jnp.bfloat16)
    cth = ct.astype(jnp.bfloat16)
    ctl = (ct - cth.astype(jnp.float32)).astype(jnp.bfloat16)
    xs = jnp.concatenate([xh, xh, xl], axis=1)
    cts = jnp.concatenate([cth, ctl, cth], axis=0)
    xc = mm(xs, cts)
    s = cn_half - xc
    iota = lax.broadcasted_iota(jnp.int32, s.shape, 1)
    big = jnp.float32(3e38)
    kbig = jnp.int32(_K)
    m1v = jnp.min(s, axis=1, keepdims=True)
    idx1 = jnp.min(jnp.where(s == m1v, iota, kbig), axis=1, keepdims=True)
    first = iota == idx1
    m2v = jnp.min(jnp.where(first, big, s), axis=1, keepdims=True)
    idx2 = jnp.min(jnp.where((s == m2v) & jnp.logical_not(first), iota, kbig),
                   axis=1, keepdims=True)
    idx12 = jnp.concatenate([idx1, idx2], axis=0)
    iota2 = lax.broadcasted_iota(jnp.int32, (2 * cc, _K), 1)
    oh12 = (iota2 == idx12).astype(jnp.bfloat16)
    ch = c.astype(jnp.bfloat16)
    cmf = c - ch.astype(jnp.float32)
    cm = cmf.astype(jnp.bfloat16)
    cl = (cmf - cm.astype(jnp.float32)).astype(jnp.bfloat16)
    c3 = jnp.concatenate([ch, cm, cl], axis=1)
    g3 = mm(oh12, c3)
    c12 = (g3[:, :_D] + g3[:, _D:2 * _D]) + g3[:, 2 * _D:]
    x12 = jnp.concatenate([x, x], axis=0)
    r = x12 - c12
    d12 = _tree_sum_minor(r * r)
    d1 = d12[:cc]
    d2 = d12[cc:]
    take1 = (d1 < d2) | ((d1 == d2) & (idx1 < idx2))
    win = jnp.where(take1, idx1, idx2)
    dmin = jnp.where(take1, d1, d2)
    idx_ref[...] = win
    q_ref[...] = jnp.where(take1, c12[:cc], c12[cc:])
    part = jnp.sum(dmin, axis=0, keepdims=True) * jnp.float32(2.0 ** -20)

    @pl.when(pl.program_id(0) == 0)
    def _():
        loss_ref[...] = jnp.zeros_like(loss_ref[...])

    loss_ref[...] += part


def _sc_gather(table, idx):
    info = plsc.get_sparse_core_info()
    nw = info.num_cores * info.num_subcores
    n = idx.shape[0]
    bpw = n // nw
    mesh = plsc.VectorSubcoreMesh(core_axis_name="c", subcore_axis_name="s")

    @functools.partial(
        pl.kernel, mesh=mesh,
        out_type=jax.ShapeDtypeStruct((n, _D), jnp.float32),
        compiler_params=pltpu.CompilerParams(use_tc_tiling_on_sc=False),
        scratch_types=[
            pltpu.VMEM((bpw,), jnp.int32),
            pltpu.VMEM((bpw, _D), jnp.float32),
            pltpu.SemaphoreType.DMA,
        ],
    )
    def gather_kernel(table_hbm, idx_hbm, out_hbm, idx_v, rows_v, sem):
        wid = lax.axis_index("s") * info.num_cores + lax.axis_index("c")
        base = wid * bpw
        pltpu.sync_copy(idx_hbm.at[pl.ds(base, bpw)], idx_v)
        pltpu.async_copy(table_hbm.at[idx_v], rows_v, sem).wait()
        pltpu.sync_copy(rows_v, out_hbm.at[pl.ds(base, bpw)])

    return gather_kernel(table, idx)


def kernel(inputs, codewords):
    shape = inputs.shape
    n = shape[0] * shape[1]
    x = inputs.reshape(n, _D)
    ct = codewords.T
    idx2d, loss2d, q2d = pl.pallas_call(
        _tc_body,
        grid=(n // _CHUNK,),
        in_specs=[
            pl.BlockSpec((_CHUNK, _D), lambda i: (i, 0)),
            pl.BlockSpec((_K, _D), lambda i: (0, 0)),
            pl.BlockSpec((_D, _K), lambda i: (0, 0)),
        ],
        out_specs=[
            pl.BlockSpec((_CHUNK, 1), lambda i: (i, 0)),
            pl.BlockSpec((1, 1), lambda i: (0, 0)),
            pl.BlockSpec((_CHUNK, _D), lambda i: (i, 0)),
        ],
        out_shape=[
            jax.ShapeDtypeStruct((n, 1), jnp.int32),
            jax.ShapeDtypeStruct((1, 1), jnp.float32),
            jax.ShapeDtypeStruct((n, _D), jnp.float32),
        ],
    )(x, codewords, ct)
    quantized = q2d.reshape(shape)
    indices = idx2d.reshape(shape[:-1])
    loss = loss2d[0, 0]
    return quantized, indices, loss


# speedup vs baseline: 8.2881x; 1.7050x over previous
import jax
import jax.numpy as jnp
from jax import lax
from jax.experimental import pallas as pl

_K = 512
_D = 32
_CHUNK = 2048


def _tree_sum_rows(v):
    gs = []
    for g in range(4):
        b = v[8 * g:8 * g + 8]
        b = b[:4] + b[4:]
        b = b[:2] + b[2:]
        b = b[:1] + b[1:]
        gs.append(b)
    return (gs[0] + gs[1]) + (gs[2] + gs[3])


def _tc_body(x_ref, c_ref, ct_ref, idx_ref, loss_ref, q_ref):
    cc = x_ref.shape[0]
    x = x_ref[...]
    c = c_ref[...]
    ct = ct_ref[...]
    cdims = (((1,), (0,)), ((), ()))

    def mm(a, b):
        return lax.dot_general(a, b, cdims,
                               preferred_element_type=jnp.float32)

    xt = jnp.transpose(x)
    xth = xt.astype(jnp.bfloat16)
    xtl = (xt - xth.astype(jnp.float32)).astype(jnp.bfloat16)
    chb = c.astype(jnp.bfloat16)
    clb = (c - chb.astype(jnp.float32)).astype(jnp.bfloat16)
    lhs_s = jnp.concatenate([chb, clb, chb], axis=1)
    rhs_s = jnp.concatenate([xth, xth, xtl], axis=0)
    cn_half = 0.5 * jnp.sum(c * c, axis=1, keepdims=True)
    st = cn_half - mm(lhs_s, rhs_s)
    iota_k = lax.broadcasted_iota(jnp.int32, st.shape, 0)
    big = jnp.float32(3e38)
    kbig = jnp.int32(_K)
    m1v = jnp.min(st, axis=0, keepdims=True)
    idx1 = jnp.min(jnp.where(st == m1v, iota_k, kbig), axis=0, keepdims=True)
    first = iota_k == idx1
    m2v = jnp.min(jnp.where(first, big, st), axis=0, keepdims=True)
    idx2 = jnp.min(jnp.where((st == m2v) & jnp.logical_not(first), iota_k,
                             kbig), axis=0, keepdims=True)
    idx12 = jnp.concatenate([idx1, idx2], axis=1)
    iota_k2 = lax.broadcasted_iota(jnp.int32, (_K, 2 * cc), 0)
    oh12 = (iota_k2 == idx12).astype(jnp.bfloat16)
    cth = ct.astype(jnp.bfloat16)
    cmf = ct - cth.astype(jnp.float32)
    cmm = cmf.astype(jnp.bfloat16)
    cll = (cmf - cmm.astype(jnp.float32)).astype(jnp.bfloat16)
    lhs_g = jnp.concatenate([cth, cmm, cll], axis=0)
    g3 = mm(lhs_g, oh12)
    c12 = (g3[:_D] + g3[_D:2 * _D]) + g3[2 * _D:]
    xt2 = jnp.concatenate([xt, xt], axis=1)
    r = xt2 - c12
    d12 = _tree_sum_rows(r * r)
    d1 = d12[:, :cc]
    d2 = d12[:, cc:]
    take1 = (d1 < d2) | ((d1 == d2) & (idx1 < idx2))
    win = jnp.where(take1, idx1, idx2)
    dmin = jnp.where(take1, d1, d2)
    idx_ref[...] = win[None]
    qt = jnp.where(take1, c12[:, :cc], c12[:, cc:])
    q_ref[...] = jnp.transpose(qt)
    part = jnp.sum(dmin, axis=1, keepdims=True) * jnp.float32(2.0 ** -20)

    @pl.when(pl.program_id(0) == 0)
    def _():
        loss_ref[...] = jnp.zeros_like(loss_ref[...])

    loss_ref[...] += part


def kernel(inputs, codewords):
    shape = inputs.shape
    n = shape[0] * shape[1]
    x = inputs.reshape(n, _D)
    ct = codewords.T
    grid = n // _CHUNK
    idx3, loss2d, q2d = pl.pallas_call(
        _tc_body,
        grid=(grid,),
        in_specs=[
            pl.BlockSpec((_CHUNK, _D), lambda i: (i, 0)),
            pl.BlockSpec((_K, _D), lambda i: (0, 0)),
            pl.BlockSpec((_D, _K), lambda i: (0, 0)),
        ],
        out_specs=[
            pl.BlockSpec((1, 1, _CHUNK), lambda i: (i, 0, 0)),
            pl.BlockSpec((1, 1), lambda i: (0, 0)),
            pl.BlockSpec((_CHUNK, _D), lambda i: (i, 0)),
        ],
        out_shape=[
            jax.ShapeDtypeStruct((grid, 1, _CHUNK), jnp.int32),
            jax.ShapeDtypeStruct((1, 1), jnp.float32),
            jax.ShapeDtypeStruct((n, _D), jnp.float32),
        ],
    )(x, codewords, ct)
    quantized = q2d.reshape(shape)
    indices = idx3.reshape(shape[:-1])
    loss = loss2d[0, 0]
    return quantized, indices, loss


# speedup vs baseline: 8.3822x; 1.0114x over previous
import jax
import jax.numpy as jnp
from jax import lax
from jax.experimental import pallas as pl

_K = 512
_D = 32
_CHUNK = 4096


def _tree_sum_rows(v):
    gs = []
    for g in range(4):
        b = v[8 * g:8 * g + 8]
        b = b[:4] + b[4:]
        b = b[:2] + b[2:]
        b = b[:1] + b[1:]
        gs.append(b)
    return (gs[0] + gs[1]) + (gs[2] + gs[3])


def _tc_body(x_ref, c_ref, ct_ref, idx_ref, loss_ref, q_ref):
    cc = x_ref.shape[0]
    x = x_ref[...]
    c = c_ref[...]
    ct = ct_ref[...]
    cdims = (((1,), (0,)), ((), ()))

    def mm(a, b):
        return lax.dot_general(a, b, cdims,
                               preferred_element_type=jnp.float32)

    xt = jnp.transpose(x)
    xth = xt.astype(jnp.bfloat16)
    xtl = (xt - xth.astype(jnp.float32)).astype(jnp.bfloat16)
    chb = c.astype(jnp.bfloat16)
    clb = (c - chb.astype(jnp.float32)).astype(jnp.bfloat16)
    lhs_s = jnp.concatenate([chb, clb, chb], axis=1)
    rhs_s = jnp.concatenate([xth, xth, xtl], axis=0)
    cn_half = 0.5 * jnp.sum(c * c, axis=1, keepdims=True)
    st = cn_half - mm(lhs_s, rhs_s)
    iota_k = lax.broadcasted_iota(jnp.int32, st.shape, 0)
    big = jnp.float32(3e38)
    kbig = jnp.int32(_K)
    m1v = jnp.min(st, axis=0, keepdims=True)
    idx1 = jnp.min(jnp.where(st == m1v, iota_k, kbig), axis=0, keepdims=True)
    first = iota_k == idx1
    m2v = jnp.min(jnp.where(first, big, st), axis=0, keepdims=True)
    idx2 = jnp.min(jnp.where((st == m2v) & jnp.logical_not(first), iota_k,
                             kbig), axis=0, keepdims=True)
    idx12 = jnp.concatenate([idx1, idx2], axis=1)
    iota_k2 = lax.broadcasted_iota(jnp.int32, (_K, 2 * cc), 0)
    oh12 = (iota_k2 == idx12).astype(jnp.bfloat16)
    cth = ct.astype(jnp.bfloat16)
    cmf = ct - cth.astype(jnp.float32)
    cmm = cmf.astype(jnp.bfloat16)
    cll = (cmf - cmm.astype(jnp.float32)).astype(jnp.bfloat16)
    lhs_g = jnp.concatenate([cth, cmm, cll], axis=0)
    g3 = mm(lhs_g, oh12)
    c12 = (g3[:_D] + g3[_D:2 * _D]) + g3[2 * _D:]
    xt2 = jnp.concatenate([xt, xt], axis=1)
    r = xt2 - c12
    d12 = _tree_sum_rows(r * r)
    d1 = d12[:, :cc]
    d2 = d12[:, cc:]
    take1 = (d1 < d2) | ((d1 == d2) & (idx1 < idx2))
    win = jnp.where(take1, idx1, idx2)
    dmin = jnp.where(take1, d1, d2)
    idx_ref[...] = win[None]
    qt = jnp.where(take1, c12[:, :cc], c12[:, cc:])
    q_ref[...] = jnp.transpose(qt)
    part = jnp.sum(dmin, axis=1, keepdims=True) * jnp.float32(2.0 ** -20)

    @pl.when(pl.program_id(0) == 0)
    def _():
        loss_ref[...] = jnp.zeros_like(loss_ref[...])

    loss_ref[...] += part


def kernel(inputs, codewords):
    shape = inputs.shape
    n = shape[0] * shape[1]
    x = inputs.reshape(n, _D)
    ct = codewords.T
    grid = n // _CHUNK
    idx3, loss2d, q2d = pl.pallas_call(
        _tc_body,
        grid=(grid,),
        in_specs=[
            pl.BlockSpec((_CHUNK, _D), lambda i: (i, 0)),
            pl.BlockSpec((_K, _D), lambda i: (0, 0)),
            pl.BlockSpec((_D, _K), lambda i: (0, 0)),
        ],
        out_specs=[
            pl.BlockSpec((1, 1, _CHUNK), lambda i: (i, 0, 0)),
            pl.BlockSpec((1, 1), lambda i: (0, 0)),
            pl.BlockSpec((_CHUNK, _D), lambda i: (i, 0)),
        ],
        out_shape=[
            jax.ShapeDtypeStruct((grid, 1, _CHUNK), jnp.int32),
            jax.ShapeDtypeStruct((1, 1), jnp.float32),
            jax.ShapeDtypeStruct((n, _D), jnp.float32),
        ],
    )(x, codewords, ct)
    quantized = q2d.reshape(shape)
    indices = idx3.reshape(shape[:-1])
    loss = loss2d[0, 0]
    return quantized, indices, loss
